# Initial kernel scaffold; baseline (speedup 1.0000x reference)
import jax
import jax.numpy as jnp
from jax.experimental import pallas as pl


def kernel(x, edge_index, batch, y, W1, b1, W2, b2, Wc, bc):
    raise NotImplementedError("write your pallas kernel here")


# speedup vs baseline: 456.5040x; 456.5040x over previous
import functools

import jax
import jax.numpy as jnp
from jax import lax
from jax.experimental import pallas as pl
from jax.experimental.pallas import tpu as pltpu
from jax.experimental.pallas import tpu_sc as plsc

N = 100000
E = 6400000
D = 128
H = 64
G = 64
FIX_R = 0.7


TN = 4000
NB = N // TN


def _tc_body(x_ref, b3_ref, w1_ref, b1_ref, w2_ref, b2_ref, wc_ref, bc_ref,
             y_ref, att_ref, clf_ref, loss_ref, seg_acc, cnt_acc, info_acc):
    i = pl.program_id(0)

    @pl.when(i == 0)
    def _init():
        seg_acc[...] = jnp.zeros_like(seg_acc)
        cnt_acc[...] = jnp.zeros_like(cnt_acc)
        info_acc[...] = jnp.zeros_like(info_acc)

    x = x_ref[...]
    h = jnp.maximum(
        jnp.dot(x, w1_ref[...], preferred_element_type=jnp.float32)
        + b1_ref[...], 0.0)
    logit = jnp.dot(h, w2_ref[...], preferred_element_type=jnp.float32) \
        + b2_ref[...]
    att = 1.0 / (1.0 + jnp.exp(-logit))
    att_ref[...] = att

    r = FIX_R
    il = att * jnp.log(att / r + 1e-06) \
        + (1.0 - att) * jnp.log((1.0 - att) / (1.0 - r + 1e-06) + 1e-06)
    info_acc[...] = info_acc[...] + jnp.sum(il).reshape(1, 1)

    bt = b3_ref[0]
    seg_ids = lax.broadcasted_iota(jnp.int32, (G, TN), 0)
    onehot = (seg_ids == bt).astype(jnp.float32)
    seg_acc[...] = seg_acc[...] + jnp.dot(
        onehot, x, preferred_element_type=jnp.float32)
    cnt_acc[...] = cnt_acc[...] + jnp.sum(onehot, axis=1, keepdims=True)

    @pl.when(i == NB - 1)
    def _fin():
        pooled = seg_acc[...] / jnp.maximum(cnt_acc[...], 1.0)
        clf = jnp.dot(pooled, wc_ref[...],
                      preferred_element_type=jnp.float32) + bc_ref[...]
        clf_ref[...] = clf
        yf = y_ref[...].astype(jnp.float32)
        pred = jnp.mean(jnp.maximum(clf, 0.0)
                        + jnp.log(1.0 + jnp.exp(-jnp.abs(clf))) - clf * yf)
        loss_ref[...] = (pred + info_acc[0, 0] / jnp.float32(N)).reshape(1, 1)


_tc_call = pl.pallas_call(
    _tc_body,
    grid=(NB,),
    in_specs=[
        pl.BlockSpec((TN, D), lambda i: (i, 0)),
        pl.BlockSpec((1, 1, TN), lambda i: (i, 0, 0)),
        pl.BlockSpec((D, H), lambda i: (0, 0)),
        pl.BlockSpec((1, H), lambda i: (0, 0)),
        pl.BlockSpec((H, 1), lambda i: (0, 0)),
        pl.BlockSpec((1, 1), lambda i: (0, 0)),
        pl.BlockSpec((D, 1), lambda i: (0, 0)),
        pl.BlockSpec((1, 1), lambda i: (0, 0)),
        pl.BlockSpec((G, 1), lambda i: (0, 0)),
    ],
    out_specs=[
        pl.BlockSpec((TN, 1), lambda i: (i, 0)),
        pl.BlockSpec((G, 1), lambda i: (0, 0)),
        pl.BlockSpec((1, 1), lambda i: (0, 0)),
    ],
    out_shape=[
        jax.ShapeDtypeStruct((N, 1), jnp.float32),
        jax.ShapeDtypeStruct((G, 1), jnp.float32),
        jax.ShapeDtypeStruct((1, 1), jnp.float32),
    ],
    scratch_shapes=[
        pltpu.VMEM((G, D), jnp.float32),
        pltpu.VMEM((G, 1), jnp.float32),
        pltpu.VMEM((1, 1), jnp.float32),
    ],
)


NC = 2
NS = 16
NW = NC * NS
EPW = E // NW
C = 8000
NCHUNK = EPW // C


def _sc_body(att_hbm, ei_hbm, out_hbm, table, idx_s, idx_d, prod):
    wid = lax.axis_index("s") * NC + lax.axis_index("c")
    pltpu.sync_copy(att_hbm, table)
    base0 = wid * EPW

    def chunk_body(ci, carry):
        base = base0 + ci * C
        pltpu.sync_copy(ei_hbm.at[pl.ds(base, C)], idx_s)
        pltpu.sync_copy(ei_hbm.at[pl.ds(E + base, C)], idx_d)

        def inner(j, carry2):
            vs = idx_s[pl.ds(j * 16, 16)]
            vd = idx_d[pl.ds(j * 16, 16)]
            a = plsc.load_gather(table, [vs])
            b = plsc.load_gather(table, [vd])
            prod[pl.ds(j * 16, 16)] = a * b
            return carry2

        lax.fori_loop(0, C // 16, inner, 0)
        pltpu.sync_copy(prod, out_hbm.at[pl.ds(base, C)])
        return carry

    lax.fori_loop(0, NCHUNK, chunk_body, 0)


@functools.cache
def _make_sc_call():
    mesh = plsc.VectorSubcoreMesh(core_axis_name="c", subcore_axis_name="s")
    return pl.kernel(
        _sc_body,
        mesh=mesh,
        compiler_params=pltpu.CompilerParams(needs_layout_passes=False),
        out_type=jax.ShapeDtypeStruct((E,), jnp.float32),
        scratch_types=[
            pltpu.VMEM((N,), jnp.float32),
            pltpu.VMEM((C,), jnp.int32),
            pltpu.VMEM((C,), jnp.int32),
            pltpu.VMEM((C,), jnp.float32),
        ],
    )


def kernel(x, edge_index, batch, y, W1, b1, W2, b2, Wc, bc):
    batch3 = batch.reshape(NB, 1, TN)
    att, clf_logits, loss = _tc_call(
        x, batch3, W1, b1.reshape(1, H), W2, b2.reshape(1, 1),
        Wc, bc.reshape(1, 1), y)
    edge_att = _make_sc_call()(att.reshape(N), edge_index.reshape(2 * E))
    return edge_att.reshape(E, 1), loss.reshape(()), clf_logits
